# baseline (speedup 1.0000x reference)
import functools

import jax
import jax.numpy as jnp
from jax import lax
from jax.experimental import pallas as pl
from jax.experimental.pallas import tpu as pltpu
from jax.experimental.pallas import tpu_sc as plsc

_BLOCK_ROWS = 1024
_CHUNK_ROWS = 64


def _tc_body(emb_ref, out_ref):
    out_ref[...] = jnp.broadcast_to(emb_ref[...][None], out_ref.shape)


def _sc_copy(seq, emb, dtype):
    info = plsc.get_sparse_core_info()
    num_workers = info.num_cores * info.num_subcores
    rows_per_worker = seq // num_workers
    n_chunks = rows_per_worker // _CHUNK_ROWS
    mesh = plsc.VectorSubcoreMesh(core_axis_name="c", subcore_axis_name="s")

    @functools.partial(
        pl.kernel,
        mesh=mesh,
        out_type=jax.ShapeDtypeStruct((seq, emb), dtype),
        scratch_types=[
            pltpu.VMEM((_CHUNK_ROWS, emb), dtype),
            pltpu.VMEM((_CHUNK_ROWS, emb), dtype),
            pltpu.SemaphoreType.DMA,
            pltpu.SemaphoreType.DMA,
            pltpu.SemaphoreType.DMA,
            pltpu.SemaphoreType.DMA,
        ],
    )
    def sc_kernel(table_hbm, out_hbm, buf0, buf1, rsem0, rsem1, wsem0, wsem1):
        wid = lax.axis_index("s") * info.num_cores + lax.axis_index("c")
        base = wid * rows_per_worker
        bufs = (buf0, buf1)
        rsems = (rsem0, rsem1)
        wsems = (wsem0, wsem1)

        def chunk_slice(i):
            return pl.ds(base + i * _CHUNK_ROWS, _CHUNK_ROWS)

        pending_write = [None, None]
        read_handles = [None] * n_chunks
        read_handles[0] = pltpu.async_copy(
            table_hbm.at[chunk_slice(0)], bufs[0], rsems[0]
        )
        for i in range(n_chunks):
            k = i % 2
            read_handles[i].wait()
            if i + 1 < n_chunks:
                k2 = (i + 1) % 2
                if pending_write[k2] is not None:
                    pending_write[k2].wait()
                    pending_write[k2] = None
                read_handles[i + 1] = pltpu.async_copy(
                    table_hbm.at[chunk_slice(i + 1)], bufs[k2], rsems[k2]
                )
            pending_write[k] = pltpu.async_copy(
                bufs[k], out_hbm.at[chunk_slice(i)], wsems[k]
            )
        for k in (0, 1):
            if pending_write[k] is not None:
                pending_write[k].wait()

    return sc_kernel


def kernel(inputs, embeddings):
    batch = inputs.shape[0]
    seq, emb = embeddings.shape
    tc_batch = batch - 1

    tc_part = pl.pallas_call(
        _tc_body,
        grid=(seq // _BLOCK_ROWS,),
        in_specs=[pl.BlockSpec((_BLOCK_ROWS, emb), lambda s: (s, 0))],
        out_specs=pl.BlockSpec((tc_batch, _BLOCK_ROWS, emb), lambda s: (0, s, 0)),
        out_shape=jax.ShapeDtypeStruct((tc_batch, seq, emb), embeddings.dtype),
    )(embeddings)

    sc_part = _sc_copy(seq, emb, embeddings.dtype)(embeddings)

    return jnp.concatenate([tc_part, sc_part[None]], axis=0)


# speedup vs baseline: 2.1058x; 2.1058x over previous
import functools

import jax
import jax.numpy as jnp
from jax import lax
from jax.experimental import pallas as pl
from jax.experimental.pallas import tpu as pltpu
from jax.experimental.pallas import tpu_sc as plsc

_CHUNK_ROWS = 64


def kernel(inputs, embeddings):
    batch = inputs.shape[0]
    seq, emb = embeddings.shape
    info = plsc.get_sparse_core_info()
    num_workers = info.num_cores * info.num_subcores
    rows_per_worker = seq // num_workers
    n_chunks = rows_per_worker // _CHUNK_ROWS

    mesh = plsc.VectorSubcoreMesh(core_axis_name="c", subcore_axis_name="s")

    @functools.partial(
        pl.kernel,
        mesh=mesh,
        out_type=jax.ShapeDtypeStruct((batch, seq, emb), embeddings.dtype),
        scratch_types=[
            pltpu.VMEM((_CHUNK_ROWS, emb), embeddings.dtype),
            pltpu.VMEM((_CHUNK_ROWS, emb), embeddings.dtype),
            pltpu.SemaphoreType.DMA,
            pltpu.SemaphoreType.DMA,
            pltpu.SemaphoreType.DMA,
            pltpu.SemaphoreType.DMA,
        ],
    )
    def sc_broadcast(table_hbm, out_hbm, buf0, buf1, rsem0, rsem1, wsem0, wsem1):
        wid = lax.axis_index("s") * info.num_cores + lax.axis_index("c")
        base = wid * rows_per_worker
        bufs = (buf0, buf1)
        rsems = (rsem0, rsem1)
        wsems = (wsem0, wsem1)

        def chunk_slice(i):
            return pl.ds(base + i * _CHUNK_ROWS, _CHUNK_ROWS)

        pending_writes = [None, None]
        read_handles = [None] * n_chunks
        read_handles[0] = pltpu.async_copy(
            table_hbm.at[chunk_slice(0)], bufs[0], rsems[0]
        )
        for i in range(n_chunks):
            k = i % 2
            read_handles[i].wait()
            if i + 1 < n_chunks:
                k2 = (i + 1) % 2
                if pending_writes[k2] is not None:
                    for h in pending_writes[k2]:
                        h.wait()
                    pending_writes[k2] = None
                read_handles[i + 1] = pltpu.async_copy(
                    table_hbm.at[chunk_slice(i + 1)], bufs[k2], rsems[k2]
                )
            pending_writes[k] = [
                pltpu.async_copy(bufs[k], out_hbm.at[b, chunk_slice(i)], wsems[k])
                for b in range(batch)
            ]
        for k in (0, 1):
            if pending_writes[k] is not None:
                for h in pending_writes[k]:
                    h.wait()

    return sc_broadcast(embeddings)
